# baseline (speedup 1.0000x reference)
import functools

import jax
import jax.numpy as jnp
from jax import lax
from jax.experimental import pallas as pl
from jax.experimental.pallas import tpu as pltpu
from jax.experimental.pallas import tpu_sc as plsc

S = 64.0
M = 0.4

_BR = 2048

_NC = 2
_NS = 16
_W = _NC * _NS


def _scale_block(x_ref, o_ref):
    o_ref[...] = x_ref[...] * S


def _margin_body(nper, nbatch, lab_hbm, data, lab_v, idx_v, val_v, sem):
    wid = lax.axis_index("s") * _NC + lax.axis_index("c")
    base = pl.multiple_of(wid * nper, 8)
    pltpu.sync_copy(lab_hbm.at[pl.ds(base, nper)], lab_v)
    for j in range(nper // 16):
        l = lab_v[pl.ds(16 * j, 16)]
        r = base + 16 * j + lax.iota(jnp.int32, 16)
        idx_v[pl.ds(16 * j, 16)] = jnp.maximum(l, 0) * nbatch + r
    pltpu.async_copy(data.at[idx_v], val_v, sem).wait()
    for j in range(nper // 16):
        l = lab_v[pl.ds(16 * j, 16)]
        delta = jnp.where(l >= 0, jnp.float32(M * S), jnp.float32(0.0))
        val_v[pl.ds(16 * j, 16)] = val_v[pl.ds(16 * j, 16)] - delta
    pltpu.async_copy(val_v, data.at[idx_v], sem).wait()


def kernel(logits, labels):
    B, C = logits.shape
    lt = logits.T

    scaled_t = pl.pallas_call(
        _scale_block,
        grid=(pl.cdiv(C, _BR),),
        in_specs=[pl.BlockSpec((_BR, B), lambda i: (i, 0))],
        out_specs=pl.BlockSpec((_BR, B), lambda i: (i, 0)),
        out_shape=jax.ShapeDtypeStruct((C, B), logits.dtype),
        compiler_params=pltpu.CompilerParams(
            dimension_semantics=("arbitrary",),
        ),
    )(lt)

    nper = B // _W
    sc_margin = pl.kernel(
        functools.partial(_margin_body, nper, B),
        out_type=(),
        mesh=plsc.VectorSubcoreMesh(
            core_axis_name="c", subcore_axis_name="s",
            num_cores=_NC, num_subcores=_NS,
        ),
        scratch_types=[
            pltpu.VMEM((nper,), jnp.int32),
            pltpu.VMEM((nper,), jnp.int32),
            pltpu.VMEM((nper,), jnp.float32),
            pltpu.SemaphoreType.DMA,
        ],
    )

    ref = jax.new_ref(scaled_t.reshape(C * B))
    sc_margin(labels, ref)
    return ref[...].reshape(C, B).T


# speedup vs baseline: 3.7287x; 3.7287x over previous
import jax
import jax.numpy as jnp
from jax.experimental import pallas as pl
from jax.experimental.pallas import tpu as pltpu

S = 64.0
M = 0.4

_BR = 1024


def _cosface_block(lab_ref, x_ref, o_ref):
    i = pl.program_id(0)
    lab = lab_ref[...]
    row = jax.lax.broadcasted_iota(jnp.int32, x_ref.shape, 0) + i * _BR
    hit = row == lab
    x = x_ref[...]
    o_ref[...] = x * S - (M * S) * hit.astype(jnp.float32)


def kernel(logits, labels):
    B, C = logits.shape
    lt = logits.T
    lab2 = labels.reshape(1, B)
    out_t = pl.pallas_call(
        _cosface_block,
        grid=(pl.cdiv(C, _BR),),
        in_specs=[
            pl.BlockSpec((1, B), lambda i: (0, 0)),
            pl.BlockSpec((_BR, B), lambda i: (i, 0)),
        ],
        out_specs=pl.BlockSpec((_BR, B), lambda i: (i, 0)),
        out_shape=jax.ShapeDtypeStruct((C, B), logits.dtype),
        compiler_params=pltpu.CompilerParams(
            dimension_semantics=("arbitrary",),
        ),
    )(lab2, lt)
    return out_t.T


# speedup vs baseline: 3.8059x; 1.0207x over previous
import jax
import jax.numpy as jnp
from jax.experimental import pallas as pl
from jax.experimental.pallas import tpu as pltpu

S = 64.0
M = 0.4

_BR = 3072


def _cosface_block(lab_ref, x_ref, o_ref):
    i = pl.program_id(0)
    lab = lab_ref[...]
    row = jax.lax.broadcasted_iota(jnp.int32, x_ref.shape, 0) + i * _BR
    hit = row == lab
    x = x_ref[...]
    o_ref[...] = x * S - (M * S) * hit.astype(jnp.float32)


def kernel(logits, labels):
    B, C = logits.shape
    lt = logits.T
    lab2 = labels.reshape(1, B)
    out_t = pl.pallas_call(
        _cosface_block,
        grid=(pl.cdiv(C, _BR),),
        in_specs=[
            pl.BlockSpec((1, B), lambda i: (0, 0)),
            pl.BlockSpec((_BR, B), lambda i: (i, 0)),
        ],
        out_specs=pl.BlockSpec((_BR, B), lambda i: (i, 0)),
        out_shape=jax.ShapeDtypeStruct((C, B), logits.dtype),
        compiler_params=pltpu.CompilerParams(
            dimension_semantics=("arbitrary",),
        ),
    )(lab2, lt)
    return out_t.T
